# baseline (speedup 1.0000x reference)
import functools

import jax
import jax.numpy as jnp
from jax import lax
from jax.experimental import pallas as pl
from jax.experimental.pallas import tpu as pltpu

_BM = 256


def _chamfer_body(src_ref, tar_ref, acc_ref, comp_ref, cham_ref, bds_ref, *, m, n):
    i = pl.program_id(1)
    nb = pl.num_programs(1)
    s = src_ref[0]
    t = tar_ref[0]

    sb = s.astype(jnp.bfloat16)
    tb = t.astype(jnp.bfloat16)
    qr = jnp.dot(sb, tb, preferred_element_type=jnp.float32)
    q2 = jnp.sum(s * s, axis=1, keepdims=True)
    r2 = jnp.sum(t * t, axis=0, keepdims=True)
    ds = (q2 + r2) - 2.0 * qr

    d0 = s[:, 0:1] - t[0:1, :]
    d1 = s[:, 1:2] - t[1:2, :]
    d2 = s[:, 2:3] - t[2:3, :]
    de = d0 * d0 + d1 * d1 + d2 * d2

    rmin = jnp.min(ds, axis=1, keepdims=True)
    jiota = lax.broadcasted_iota(jnp.int32, (ds.shape[0], n), 1)
    jidx = jnp.where(ds == rmin, jiota, n)
    jstar = jnp.min(jidx, axis=1, keepdims=True)
    accq = jnp.sum(jnp.where(jidx == jstar, de, 0.0), axis=1, keepdims=True)
    accv = jnp.sqrt(accq)
    acc_ref[0, 0] = accv

    cmin = jnp.min(ds, axis=0, keepdims=True)
    riota = lax.broadcasted_iota(jnp.int32, (ds.shape[0], n), 0)
    iidx = jnp.where(ds == cmin, riota, ds.shape[0])
    istar = jnp.min(iidx, axis=0, keepdims=True)
    cde = jnp.sum(jnp.where(iidx == istar, de, 0.0), axis=0, keepdims=True)

    @pl.when(i == 0)
    def _init():
        bds_ref[...] = cmin
        comp_ref[0] = cde
        cham_ref[0, 0, :] = jnp.sum(accv).reshape(1)

    @pl.when(i > 0)
    def _accum():
        upd = cmin < bds_ref[...]
        bds_ref[...] = jnp.where(upd, cmin, bds_ref[...])
        comp_ref[0] = jnp.where(upd, cde, comp_ref[0])
        cham_ref[0, 0, :] = cham_ref[0, 0, :] + jnp.sum(accv)

    @pl.when(i == nb - 1)
    def _finish():
        comp = jnp.sqrt(comp_ref[0, 0, :])
        comp_ref[0, 0, :] = comp
        cham_ref[0, 0, :] = 0.5 * (cham_ref[0, 0, :] / m + jnp.sum(comp) / n)


def kernel(tar, src):
    b, n, _ = tar.shape
    m = src.shape[1]
    tar_t = jnp.transpose(tar, (0, 2, 1))

    acc, comp, cham = pl.pallas_call(
        functools.partial(_chamfer_body, m=m, n=n),
        grid=(b, m // _BM),
        in_specs=[
            pl.BlockSpec((1, _BM, 3), lambda b_, i: (b_, i, 0)),
            pl.BlockSpec((1, 3, n), lambda b_, i: (b_, 0, 0)),
        ],
        out_specs=[
            pl.BlockSpec((1, 1, _BM, 1), lambda b_, i: (b_, i, 0, 0)),
            pl.BlockSpec((1, 1, n), lambda b_, i: (b_, 0, 0)),
            pl.BlockSpec((1, 1, 1), lambda b_, i: (b_, 0, 0)),
        ],
        out_shape=[
            jax.ShapeDtypeStruct((b, m // _BM, _BM, 1), jnp.float32),
            jax.ShapeDtypeStruct((b, 1, n), jnp.float32),
            jax.ShapeDtypeStruct((b, 1, 1), jnp.float32),
        ],
        scratch_shapes=[pltpu.VMEM((1, n), jnp.float32)],
    )(src, tar_t)
    return (acc.reshape(b, m), comp[:, 0, :], cham[:, 0, 0])
